# baseline (speedup 1.0000x reference)
import functools

import jax
import jax.numpy as jnp
from jax import lax
from jax.experimental import pallas as pl
from jax.experimental.pallas import tpu as pltpu
from jax.experimental.pallas import tpu_sc as plsc

_L = 16
_K = 16
_SCR = 12


def _sc_geometry():
    try:
        info = plsc.get_sparse_core_info()
        return info.num_cores, info.num_subcores
    except Exception:
        return 2, 16


@functools.lru_cache(maxsize=None)
def _build(nq, nk):
    nc, ns = _sc_geometry()
    nw = nc * ns
    nu = nk // 2
    assert nu & (nu - 1) == 0, nu
    steps = (nu - 1).bit_length()
    qpw = nq // nw
    assert qpw * nw == nq, (nq, nw)
    chunk = min(4096, qpw)
    nch = qpw // chunk
    assert nch * chunk == qpw and nch % 2 == 0, (qpw, chunk)
    nvec = chunk // _L
    assert nvec % _K == 0, (nvec, _K)

    mesh = plsc.VectorSubcoreMesh(
        core_axis_name="c", subcore_axis_name="s", num_cores=nc,
        num_subcores=ns)

    def scr(j):
        return j ^ (j >> _SCR)

    def body(t_hbm, teven_hbm, vdata_hbm, vprev_hbm, todd_hbm, out_hbm,
             u_v, q0, q1, j0b, j1b, e0, e1, iv0, iv1,
             w0, w1, va0, va1, vb0, vb1,
             sq0, sq1, se0, se1, sv0, sv1, so0, so1):
        qs, js, es = (q0, q1), (j0b, j1b), (e0, e1)
        ivs, ws = (iv0, iv1), (w0, w1)
        vas, vbs = (va0, va1), (vb0, vb1)
        sqs, ses, svs, sos = (sq0, sq1), (se0, se1), (sv0, sv1), (so0, so1)

        sid = lax.axis_index("s")
        wid = sid * nc + lax.axis_index("c")
        base = wid * qpw

        pltpu.sync_copy(todd_hbm, u_v)

        def off(i):
            return base + i * chunk

        def stage_a(i, p):
            pltpu.make_async_copy(
                t_hbm.at[pl.ds(off(i), chunk)], qs[p], sqs[p]).wait()

            def pass_a(ib, c):
                sls = [pl.ds((ib * _K + k) * _L, _L) for k in range(_K)]
                tq = [qs[p][sls[k]] for k in range(_K)]
                cc = [jnp.zeros((_L,), jnp.int32) for _ in range(_K)]
                for b in range(steps - 1, -1, -1):
                    for k in range(_K):
                        mid = cc[k] + ((1 << b) - 1)
                        lt = plsc.load_gather(u_v, [scr(mid)]) < tq[k]
                        cc[k] = jnp.where(lt, cc[k] + (1 << b), cc[k])
                for k in range(_K):
                    js[p][sls[k]] = cc[k]
                return c

            lax.fori_loop(0, nvec // _K, pass_a, 0)
            pltpu.async_copy(teven_hbm.at[js[p]], es[p], ses[p])

        def stage_b(i, p):
            pltpu.make_async_copy(teven_hbm.at[js[p]], es[p], ses[p]).wait()

            @pl.when(i >= 3)
            def _wait_prev_store():
                pltpu.make_async_copy(
                    ws[p], out_hbm.at[pl.ds(off(i - 3), chunk)],
                    sos[p]).wait()

            def pass_b(ib, c):
                sl = pl.ds(ib * _L, _L)
                tq = qs[p][sl]
                e = es[p][sl]
                j0 = js[p][sl]
                jm = jnp.maximum(j0 - 1, 0)
                um = plsc.load_gather(u_v, [scr(jm)])
                up = plsc.load_gather(u_v, [scr(j0)])
                j2 = j0 << 1
                even = (e >= tq) & (j0 > 0)
                idx = jnp.maximum(jnp.where(even, j2, j2 + 1), 1)
                t0 = jnp.where(even, um, e)
                t1 = jnp.where(even, e, up)
                w = (tq - t0) / (t1 - t0)
                ws[p][sl] = jnp.minimum(jnp.maximum(w, 0.0), 1.0)
                ivs[p][sl] = idx
                return c

            lax.fori_loop(0, nvec, pass_b, 0)
            pltpu.async_copy(vprev_hbm.at[ivs[p]], vas[p], svs[p])
            pltpu.async_copy(vdata_hbm.at[ivs[p]], vbs[p], svs[p])

        def stage_c(i, p):
            pltpu.make_async_copy(vprev_hbm.at[ivs[p]], vas[p], svs[p]).wait()
            pltpu.make_async_copy(vdata_hbm.at[ivs[p]], vbs[p], svs[p]).wait()

            def pass_c(ib, c):
                sl = pl.ds(ib * _L, _L)
                va = vas[p][sl]
                ws[p][sl] = va + ws[p][sl] * (vbs[p][sl] - va)
                return c

            lax.fori_loop(0, nvec, pass_c, 0)
            pltpu.async_copy(ws[p], out_hbm.at[pl.ds(off(i - 2), chunk)],
                             sos[p])

        pltpu.async_copy(t_hbm.at[pl.ds(off(0), chunk)], qs[0], sqs[0])

        def pipe_body(ii, carry):
            for par in (0, 1):
                i = ii * 2 + par

                @pl.when(i < nch)
                def _a():
                    stage_a(i, par)

                @pl.when((i >= 1) & (i <= nch))
                def _b():
                    stage_b(i, 1 - par)

                @pl.when(i + 1 < nch)
                def _q():
                    pltpu.async_copy(
                        t_hbm.at[pl.ds(off(i + 1), chunk)], qs[1 - par],
                        sqs[1 - par])

                @pl.when(i >= 2)
                def _c():
                    stage_c(i, par)

            return carry

        lax.fori_loop(0, (nch + 2) // 2, pipe_body, 0)
        for c in (nch - 2, nch - 1):
            pltpu.make_async_copy(
                ws[c % 2], out_hbm.at[pl.ds(off(c), chunk)],
                sos[c % 2]).wait()

    f32 = jnp.float32
    i32 = jnp.int32
    dbl = lambda shape, dt: [pltpu.VMEM(shape, dt), pltpu.VMEM(shape, dt)]
    return pl.kernel(
        body,
        out_type=jax.ShapeDtypeStruct((nq,), f32),
        mesh=mesh,
        scratch_types=[
            pltpu.VMEM((nu,), f32),
            *dbl((chunk,), f32),
            *dbl((chunk,), i32),
            *dbl((chunk,), f32),
            *dbl((chunk,), i32),
            *dbl((chunk,), f32),
            *dbl((chunk,), f32),
            *dbl((chunk,), f32),
            *[pltpu.SemaphoreType.DMA] * 8,
        ],
        compiler_params=pltpu.CompilerParams(needs_layout_passes=False),
    )


def kernel(t, t_data, v_data):
    nq = t.shape[0]
    nk = t_data.shape[0]
    nu = nk // 2
    todd = t_data[1::2]
    jj = jnp.arange(nu, dtype=jnp.int32)
    todd_scr = jnp.zeros_like(todd).at[jj ^ (jj >> _SCR)].set(todd)
    teven = t_data[0::2]
    vprev = jnp.roll(v_data, 1)
    return _build(nq, nk)(t, teven, v_data, vprev, todd_scr)


# speedup vs baseline: 1.4589x; 1.4589x over previous
import functools

import jax
import jax.numpy as jnp
from jax import lax
from jax.experimental import pallas as pl
from jax.experimental.pallas import tpu as pltpu
from jax.experimental.pallas import tpu_sc as plsc

_L = 16
_K = 16
_SCR = 12


def _sc_geometry():
    try:
        info = plsc.get_sparse_core_info()
        return info.num_cores, info.num_subcores
    except Exception:
        return 2, 16


@functools.lru_cache(maxsize=None)
def _build(nq, nk):
    nc, ns = _sc_geometry()
    nw = nc * ns
    nu = nk // 2
    assert nu & (nu - 1) == 0, nu
    steps = (nu - 1).bit_length()
    qpw = nq // nw
    assert qpw * nw == nq, (nq, nw)
    chunk = min(2048, qpw)
    nch = qpw // chunk
    assert nch * chunk == qpw and nch % 2 == 0, (qpw, chunk)
    nvec = chunk // _L
    assert nvec % _K == 0, (nvec, _K)

    mesh = plsc.VectorSubcoreMesh(
        core_axis_name="c", subcore_axis_name="s", num_cores=nc,
        num_subcores=ns)

    def scr(j):
        return j ^ (j >> _SCR)

    def body(t_hbm, teven_hbm, vdata_hbm, vprev_hbm, todd_hbm, out_hbm,
             u_v, q0, q1, j0b, j1b, e0, e1, iv0, iv1,
             w0, w1, va0, va1, vb0, vb1, te_s, vd_s, vp_s,
             sq0, sq1, se0, se1, sv0, sv1, so0, so1):
        qs, js, es = (q0, q1), (j0b, j1b), (e0, e1)
        ivs, ws = (iv0, iv1), (w0, w1)
        vas, vbs = (va0, va1), (vb0, vb1)
        sqs, ses, svs, sos = (sq0, sq1), (se0, se1), (sv0, sv1), (so0, so1)

        sid = lax.axis_index("s")
        wid = sid * nc + lax.axis_index("c")
        base = wid * qpw

        @pl.when(sid == 0)
        def _stage():
            pltpu.sync_copy(teven_hbm, te_s)
            pltpu.sync_copy(vdata_hbm, vd_s)
            pltpu.sync_copy(vprev_hbm, vp_s)

        pltpu.sync_copy(todd_hbm, u_v)
        plsc.subcore_barrier()

        def off(i):
            return base + i * chunk

        def stage_a(i, p):
            pltpu.make_async_copy(
                t_hbm.at[pl.ds(off(i), chunk)], qs[p], sqs[p]).wait()

            def pass_a(ib, c):
                sls = [pl.ds((ib * _K + k) * _L, _L) for k in range(_K)]
                tq = [qs[p][sls[k]] for k in range(_K)]
                cc = [jnp.zeros((_L,), jnp.int32) for _ in range(_K)]
                for b in range(steps - 1, -1, -1):
                    for k in range(_K):
                        mid = cc[k] + ((1 << b) - 1)
                        lt = plsc.load_gather(u_v, [scr(mid)]) < tq[k]
                        cc[k] = jnp.where(lt, cc[k] + (1 << b), cc[k])
                for k in range(_K):
                    js[p][sls[k]] = cc[k]
                return c

            lax.fori_loop(0, nvec // _K, pass_a, 0)
            pltpu.async_copy(te_s.at[js[p]], es[p], ses[p])

        def stage_b(i, p):
            pltpu.make_async_copy(te_s.at[js[p]], es[p], ses[p]).wait()

            @pl.when(i >= 3)
            def _wait_prev_store():
                pltpu.make_async_copy(
                    ws[p], out_hbm.at[pl.ds(off(i - 3), chunk)],
                    sos[p]).wait()

            def pass_b(ib, c):
                sl = pl.ds(ib * _L, _L)
                tq = qs[p][sl]
                e = es[p][sl]
                j0 = js[p][sl]
                jm = jnp.maximum(j0 - 1, 0)
                um = plsc.load_gather(u_v, [scr(jm)])
                up = plsc.load_gather(u_v, [scr(j0)])
                j2 = j0 << 1
                even = (e >= tq) & (j0 > 0)
                idx = jnp.maximum(jnp.where(even, j2, j2 + 1), 1)
                t0 = jnp.where(even, um, e)
                t1 = jnp.where(even, e, up)
                w = (tq - t0) / (t1 - t0)
                ws[p][sl] = jnp.minimum(jnp.maximum(w, 0.0), 1.0)
                ivs[p][sl] = idx
                return c

            lax.fori_loop(0, nvec, pass_b, 0)
            pltpu.async_copy(vp_s.at[ivs[p]], vas[p], svs[p])
            pltpu.async_copy(vd_s.at[ivs[p]], vbs[p], svs[p])

        def stage_c(i, p):
            pltpu.make_async_copy(vp_s.at[ivs[p]], vas[p], svs[p]).wait()
            pltpu.make_async_copy(vd_s.at[ivs[p]], vbs[p], svs[p]).wait()

            def pass_c(ib, c):
                sl = pl.ds(ib * _L, _L)
                va = vas[p][sl]
                ws[p][sl] = va + ws[p][sl] * (vbs[p][sl] - va)
                return c

            lax.fori_loop(0, nvec, pass_c, 0)
            pltpu.async_copy(ws[p], out_hbm.at[pl.ds(off(i - 2), chunk)],
                             sos[p])

        pltpu.async_copy(t_hbm.at[pl.ds(off(0), chunk)], qs[0], sqs[0])

        def pipe_body(ii, carry):
            for par in (0, 1):
                i = ii * 2 + par

                @pl.when(i < nch)
                def _a():
                    stage_a(i, par)

                @pl.when((i >= 1) & (i <= nch))
                def _b():
                    stage_b(i, 1 - par)

                @pl.when(i + 1 < nch)
                def _q():
                    pltpu.async_copy(
                        t_hbm.at[pl.ds(off(i + 1), chunk)], qs[1 - par],
                        sqs[1 - par])

                @pl.when(i >= 2)
                def _c():
                    stage_c(i, par)

            return carry

        lax.fori_loop(0, (nch + 2) // 2, pipe_body, 0)
        for c in (nch - 2, nch - 1):
            pltpu.make_async_copy(
                ws[c % 2], out_hbm.at[pl.ds(off(c), chunk)],
                sos[c % 2]).wait()

    f32 = jnp.float32
    i32 = jnp.int32
    dbl = lambda shape, dt: [pltpu.VMEM(shape, dt), pltpu.VMEM(shape, dt)]
    return pl.kernel(
        body,
        out_type=jax.ShapeDtypeStruct((nq,), f32),
        mesh=mesh,
        scratch_types=[
            pltpu.VMEM((nu,), f32),
            *dbl((chunk,), f32),
            *dbl((chunk,), i32),
            *dbl((chunk,), f32),
            *dbl((chunk,), i32),
            *dbl((chunk,), f32),
            *dbl((chunk,), f32),
            *dbl((chunk,), f32),
            pltpu.VMEM_SHARED((nu,), f32),
            pltpu.VMEM_SHARED((nk,), f32),
            pltpu.VMEM_SHARED((nk,), f32),
            *[pltpu.SemaphoreType.DMA] * 8,
        ],
        compiler_params=pltpu.CompilerParams(needs_layout_passes=False),
    )


def kernel(t, t_data, v_data):
    nq = t.shape[0]
    nk = t_data.shape[0]
    nu = nk // 2
    todd = t_data[1::2]
    jj = jnp.arange(nu, dtype=jnp.int32)
    todd_scr = jnp.zeros_like(todd).at[jj ^ (jj >> _SCR)].set(todd)
    teven = t_data[0::2]
    vprev = jnp.roll(v_data, 1)
    return _build(nq, nk)(t, teven, v_data, vprev, todd_scr)


# speedup vs baseline: 1.4849x; 1.0178x over previous
import functools

import jax
import jax.numpy as jnp
from jax import lax
from jax.experimental import pallas as pl
from jax.experimental.pallas import tpu as pltpu
from jax.experimental.pallas import tpu_sc as plsc

_L = 16
_K = 16


def _sc_geometry():
    try:
        info = plsc.get_sparse_core_info()
        return info.num_cores, info.num_subcores
    except Exception:
        return 2, 16


@functools.lru_cache(maxsize=None)
def _build(nq, nk):
    nc, ns = _sc_geometry()
    nw = nc * ns
    nu = nk // 2
    assert nu & (nu - 1) == 0, nu
    steps = (nu - 1).bit_length()
    qpw = nq // nw
    assert qpw * nw == nq, (nq, nw)
    chunk = min(2048, qpw)
    nch = qpw // chunk
    assert nch * chunk == qpw and nch % 2 == 0, (qpw, chunk)
    nvec = chunk // _L
    assert nvec % _K == 0, (nvec, _K)

    mesh = plsc.VectorSubcoreMesh(
        core_axis_name="c", subcore_axis_name="s", num_cores=nc,
        num_subcores=ns)

    def body(t_hbm, teven_hbm, vdata_hbm, todd_hbm, out_hbm,
             u_v, q0, q1, j0b, j1b, e0, e1, im0, im1, iv0, iv1,
             w0, w1, va0, va1, vb0, vb1, o0, o1, te_s, vd_s,
             sq0, sq1, se0, se1, sv0, sv1, so0, so1):
        qs, js, es = (q0, q1), (j0b, j1b), (e0, e1)
        ims, ivs, ws = (im0, im1), (iv0, iv1), (w0, w1)
        vas, vbs, os_ = (va0, va1), (vb0, vb1), (o0, o1)
        sqs, ses, svs, sos = (sq0, sq1), (se0, se1), (sv0, sv1), (so0, so1)

        sid = lax.axis_index("s")
        wid = sid * nc + lax.axis_index("c")
        base = wid * qpw

        @pl.when(sid == 0)
        def _stage():
            pltpu.sync_copy(teven_hbm, te_s)
            pltpu.sync_copy(vdata_hbm, vd_s)

        pltpu.sync_copy(todd_hbm, u_v)
        plsc.subcore_barrier()

        def off(i):
            return base + i * chunk

        def stage_a(i, p):
            pltpu.make_async_copy(
                t_hbm.at[pl.ds(off(i), chunk)], qs[p], sqs[p]).wait()

            def pass_a(ib, c):
                sls = [pl.ds((ib * _K + k) * _L, _L) for k in range(_K)]
                tq = [qs[p][sls[k]] for k in range(_K)]
                cc = [jnp.zeros((_L,), jnp.int32) for _ in range(_K)]
                for b in range(steps - 1, -1, -1):
                    for k in range(_K):
                        mid = cc[k] + ((1 << b) - 1)
                        addr = mid ^ (mid >> 12)
                        lt = plsc.load_gather(u_v, [addr]) < tq[k]
                        cc[k] = jnp.where(lt, cc[k] + (1 << b), cc[k])
                for k in range(_K):
                    js[p][sls[k]] = cc[k]
                return c

            lax.fori_loop(0, nvec // _K, pass_a, 0)
            pltpu.async_copy(te_s.at[js[p]], es[p], ses[p])

        def stage_b(i, p):
            pltpu.make_async_copy(te_s.at[js[p]], es[p], ses[p]).wait()

            def pass_b(ib, c):
                sl = pl.ds(ib * _L, _L)
                tq = qs[p][sl]
                e = es[p][sl]
                j0 = js[p][sl]
                jm = jnp.maximum(j0 - 1, 0)
                um = plsc.load_gather(u_v, [jm ^ (jm >> 12)])
                up = plsc.load_gather(u_v, [j0 ^ (j0 >> 12)])
                j2 = j0 << 1
                even = (e >= tq) & (j0 > 0)
                idx = jnp.maximum(jnp.where(even, j2, j2 + 1), 1)
                t0 = jnp.where(even, um, e)
                t1 = jnp.where(even, e, up)
                w = (tq - t0) / (t1 - t0)
                ws[p][sl] = jnp.minimum(jnp.maximum(w, 0.0), 1.0)
                ims[p][sl] = idx - 1
                ivs[p][sl] = idx
                return c

            lax.fori_loop(0, nvec, pass_b, 0)
            pltpu.async_copy(vd_s.at[ims[p]], vas[p], svs[p])
            pltpu.async_copy(vd_s.at[ivs[p]], vbs[p], svs[p])

        def stage_c(i, p):
            pltpu.make_async_copy(vd_s.at[ims[p]], vas[p], svs[p]).wait()
            pltpu.make_async_copy(vd_s.at[ivs[p]], vbs[p], svs[p]).wait()

            @pl.when(i >= 4)
            def _wait_prev_store():
                pltpu.make_async_copy(
                    os_[p], out_hbm.at[pl.ds(off(i - 4), chunk)],
                    sos[p]).wait()

            def pass_c(ib, c):
                sl = pl.ds(ib * _L, _L)
                os_[p][sl] = vas[p][sl] + ws[p][sl] * (vbs[p][sl] - vas[p][sl])
                return c

            lax.fori_loop(0, nvec, pass_c, 0)
            pltpu.async_copy(os_[p], out_hbm.at[pl.ds(off(i - 2), chunk)],
                             sos[p])

        pltpu.async_copy(t_hbm.at[pl.ds(off(0), chunk)], qs[0], sqs[0])

        def pipe_body(ii, carry):
            for par in (0, 1):
                i = ii * 2 + par

                @pl.when(i < nch)
                def _a():
                    stage_a(i, par)

                @pl.when((i >= 1) & (i <= nch))
                def _b():
                    stage_b(i, 1 - par)

                @pl.when(i + 1 < nch)
                def _q():
                    pltpu.async_copy(
                        t_hbm.at[pl.ds(off(i + 1), chunk)], qs[1 - par],
                        sqs[1 - par])

                @pl.when(i >= 2)
                def _c():
                    stage_c(i, par)

            return carry

        lax.fori_loop(0, (nch + 2) // 2, pipe_body, 0)
        for c in (nch - 2, nch - 1):
            pltpu.make_async_copy(
                os_[c % 2], out_hbm.at[pl.ds(off(c), chunk)],
                sos[c % 2]).wait()

    f32 = jnp.float32
    i32 = jnp.int32
    dbl = lambda shape, dt: [pltpu.VMEM(shape, dt), pltpu.VMEM(shape, dt)]
    return pl.kernel(
        body,
        out_type=jax.ShapeDtypeStruct((nq,), f32),
        mesh=mesh,
        scratch_types=[
            pltpu.VMEM((nu,), f32),
            *dbl((chunk,), f32),
            *dbl((chunk,), i32),
            *dbl((chunk,), f32),
            *dbl((chunk,), i32),
            *dbl((chunk,), i32),
            *dbl((chunk,), f32),
            *dbl((chunk,), f32),
            *dbl((chunk,), f32),
            *dbl((chunk,), f32),
            pltpu.VMEM_SHARED((nu,), f32),
            pltpu.VMEM_SHARED((nk,), f32),
            *[pltpu.SemaphoreType.DMA] * 8,
        ],
        compiler_params=pltpu.CompilerParams(needs_layout_passes=False),
    )


def kernel(t, t_data, v_data):
    nq = t.shape[0]
    nk = t_data.shape[0]
    todd = t_data[1::2]
    nu = nk // 2
    jj = jnp.arange(nu, dtype=jnp.int32)
    todd_scr = jnp.zeros_like(todd).at[jj ^ (jj >> 12)].set(todd)
    teven = t_data[0::2]
    return _build(nq, nk)(t, teven, v_data, todd_scr)
